# speedup vs baseline: 4.3903x; 4.3903x over previous
import jax
import jax.numpy as jnp
from jax.experimental import pallas as pl

_N_SUBS = 64
_D_MID = 64
_D_OUT = 64


def _pass1_body(x_ref, sub_ref, r_ref, st_ref):
    x = x_ref[...]
    sub = sub_ref[0, 0, :]
    r = jnp.sum(x, axis=1)
    r_ref[0, 0, :] = r
    b = x.shape[0]
    seg = jax.lax.broadcasted_iota(jnp.int32, (b, _N_SUBS), 1)
    mask = (sub[:, None] == seg).astype(jnp.float32)
    s_c = jnp.sum(mask * r[:, None], axis=0, keepdims=True)
    c_c = jnp.sum(mask, axis=0, keepdims=True)

    @pl.when(pl.program_id(0) == 0)
    def _init():
        st_ref[...] = jnp.zeros_like(st_ref)

    st_ref[0:1, :] += s_c
    st_ref[1:2, :] += c_c


def _pass2_body(g_ref, l_ref, sub_ref, r_ref, r0_ref, st_ref, out_ref):
    S = st_ref[0:1, :]
    C = st_ref[1:2, :]
    r0 = r0_ref[0, 0, 0]
    m = jnp.where(C > 0, S / jnp.maximum(C, 1.0), r0)
    t = jnp.maximum(g_ref[0:1, 0:64] * m, 0.0)
    sub = sub_ref[0, 0, :]
    b = sub.shape[0]
    seg = jax.lax.broadcasted_iota(jnp.int32, (b, _N_SUBS), 1)
    mask = sub[:, None] == seg
    g = jnp.sum(jnp.where(mask, t, 0.0), axis=1, keepdims=True)
    r = r_ref[0, 0, :][:, None]
    lam = l_ref[0:1, 0:1]
    v = jnp.maximum(lam * (r + _D_MID * g), 0.0)
    out_ref[...] = jnp.broadcast_to(v, (b, _D_OUT))


def kernel(x, sub, Gamma, Lambda):
    n, d_in = x.shape
    B = 8000
    nb = n // B
    sub3 = sub.reshape(nb, 1, B)
    gv = jnp.broadcast_to(Gamma.reshape(1, 1), (8, _N_SUBS))
    lv = jnp.broadcast_to(Lambda.reshape(1, 1), (8, _N_SUBS))

    r3, st = pl.pallas_call(
        _pass1_body,
        grid=(nb,),
        in_specs=[
            pl.BlockSpec((B, d_in), lambda i: (i, 0)),
            pl.BlockSpec((1, 1, B), lambda i: (i, 0, 0)),
        ],
        out_specs=[
            pl.BlockSpec((1, 1, B), lambda i: (i, 0, 0)),
            pl.BlockSpec((8, _N_SUBS), lambda i: (0, 0)),
        ],
        out_shape=[
            jax.ShapeDtypeStruct((nb, 1, B), jnp.float32),
            jax.ShapeDtypeStruct((8, _N_SUBS), jnp.float32),
        ],
    )(x, sub3)

    out = pl.pallas_call(
        _pass2_body,
        grid=(nb,),
        in_specs=[
            pl.BlockSpec((8, _N_SUBS), lambda i: (0, 0)),
            pl.BlockSpec((8, _N_SUBS), lambda i: (0, 0)),
            pl.BlockSpec((1, 1, B), lambda i: (i, 0, 0)),
            pl.BlockSpec((1, 1, B), lambda i: (i, 0, 0)),
            pl.BlockSpec((1, 1, B), lambda i: (0, 0, 0)),
            pl.BlockSpec((8, _N_SUBS), lambda i: (0, 0)),
        ],
        out_specs=pl.BlockSpec((B, _D_OUT), lambda i: (i, 0)),
        out_shape=jax.ShapeDtypeStruct((n, _D_OUT), jnp.float32),
    )(gv, lv, sub3, r3, r3, st)
    return out


# speedup vs baseline: 7.3659x; 1.6778x over previous
import jax
import jax.numpy as jnp
from jax.experimental import pallas as pl

_N_SUBS = 64
_D_MID = 64
_D_OUT = 64


def _pass1_body(x_ref, sub_ref, st_ref):
    x = x_ref[...]
    b, d_in = x.shape
    xh = x.astype(jnp.bfloat16)
    sub = sub_ref[0, 0, :]
    segT = jax.lax.broadcasted_iota(jnp.int32, (_N_SUBS, b), 0)
    maskT = sub[None, :] == segT
    mseg = jax.lax.dot_general(
        maskT.astype(jnp.bfloat16), xh, (((1,), (0,)), ((), ())),
        preferred_element_type=jnp.float32)
    cnt = jnp.sum(maskT.astype(jnp.float32), axis=1, keepdims=True)

    @pl.when(pl.program_id(0) == 0)
    def _init():
        st_ref[...] = jnp.zeros_like(st_ref)

    st_ref[:, :d_in] += mseg
    st_ref[:, d_in:d_in + 1] += cnt

    @pl.when(pl.program_id(0) == 0)
    def _stash_r0():
        st_ref[0:1, d_in + 1:d_in + 2] = jnp.sum(
            x[0:1, :], axis=1, keepdims=True)


def _pass2_body(g_ref, l_ref, x_ref, sub_ref, st_ref, out_ref):
    x = x_ref[...]
    b, d_in = x.shape
    S = jnp.sum(st_ref[:, :d_in], axis=1)
    C = st_ref[:, d_in]

    xh = x.astype(jnp.bfloat16)
    xl = (x - xh.astype(jnp.float32)).astype(jnp.bfloat16)
    ones = jnp.ones((d_in, _D_OUT), jnp.bfloat16)
    dot = lambda a, c: jax.lax.dot_general(
        a, c, (((1,), (0,)), ((), ())), preferred_element_type=jnp.float32)
    rB = dot(xh, ones) + dot(xl, ones)

    r0 = st_ref[0, d_in + 1]
    m = jnp.where(C > 0, S / jnp.maximum(C, 1.0), r0)
    gamma = g_ref[0, 0]
    t = jnp.maximum(gamma * m, 0.0) * _D_MID
    t2 = jnp.broadcast_to(t[:, None], (_N_SUBS, _N_SUBS))
    t2 = t2.astype(jnp.bfloat16)

    sub = sub_ref[0, 0, :]
    seg = jax.lax.broadcasted_iota(jnp.int32, (b, _N_SUBS), 1)
    mask = (sub[:, None] == seg).astype(jnp.bfloat16)
    gB = dot(mask, t2)
    lam = l_ref[0, 0]
    out_ref[...] = jnp.maximum(lam * (rB + gB), 0.0)


def kernel(x, sub, Gamma, Lambda):
    n, d_in = x.shape
    B = 16000
    nb = n // B
    sub3 = sub.reshape(nb, 1, B)
    gv = jnp.broadcast_to(Gamma.reshape(1, 1), (8, 128))
    lv = jnp.broadcast_to(Lambda.reshape(1, 1), (8, 128))

    st = pl.pallas_call(
        _pass1_body,
        grid=(nb,),
        in_specs=[
            pl.BlockSpec((B, d_in), lambda i: (i, 0)),
            pl.BlockSpec((1, 1, B), lambda i: (i, 0, 0)),
        ],
        out_specs=pl.BlockSpec((_N_SUBS, d_in + 2 * _N_SUBS), lambda i: (0, 0)),
        out_shape=jax.ShapeDtypeStruct((_N_SUBS, d_in + 2 * _N_SUBS), jnp.float32),
    )(x, sub3)

    out = pl.pallas_call(
        _pass2_body,
        grid=(nb,),
        in_specs=[
            pl.BlockSpec((8, 128), lambda i: (0, 0)),
            pl.BlockSpec((8, 128), lambda i: (0, 0)),
            pl.BlockSpec((B, d_in), lambda i: (i, 0)),
            pl.BlockSpec((1, 1, B), lambda i: (i, 0, 0)),
            pl.BlockSpec((_N_SUBS, d_in + 2 * _N_SUBS), lambda i: (0, 0)),
        ],
        out_specs=pl.BlockSpec((B, _D_OUT), lambda i: (i, 0)),
        out_shape=jax.ShapeDtypeStruct((n, _D_OUT), jnp.float32),
    )(gv, lv, x, sub3, st)
    return out
